# baseline (speedup 1.0000x reference)
import functools

import jax
import jax.numpy as jnp
from jax import lax
from jax.experimental import pallas as pl
from jax.experimental.pallas import tpu as pltpu
from jax.experimental.pallas import tpu_sc as plsc

_K = 20
_KPAD = 32
_NEG = -3.0e38
_LANES = 128


def _score_topk_kernel(bags_ref, w_enc_ref, b_enc_ref, w_att_ref, b_att_ref,
                       idx_ref, w_ref, scores_ref, *, bb, nb_blk, n_total, k):
    b_i = pl.program_id(0)
    n_i = pl.program_id(1)
    num_n = pl.num_programs(1)

    x = bags_ref[...]
    f = x.shape[-1]
    x2 = x.reshape(bb * nb_blk, f)
    emb = jnp.dot(x2, w_enc_ref[...], preferred_element_type=jnp.float32)
    emb = jnp.maximum(emb + b_enc_ref[...], 0.0)
    s = jnp.dot(emb, w_att_ref[...],
                preferred_element_type=jnp.float32)[:, 0] + b_att_ref[0, 0]
    scores_ref[n_i] = s.reshape(bb, nb_blk)

    @pl.when(n_i == num_n - 1)
    def _finalize():
        cur = jnp.concatenate([scores_ref[c] for c in range(num_n)],
                              axis=1)
        iota = lax.broadcasted_iota(jnp.int32, (bb, n_total), 1)
        kiota = lax.broadcasted_iota(jnp.int32, (bb, _LANES), 1)

        def body(i, carry):
            cur, vmax, denom, idxs, wacc = carry
            m = jnp.max(cur, axis=1, keepdims=True)
            cand = jnp.where(cur == m, iota, n_total)
            idx = jnp.min(cand, axis=1, keepdims=True)
            cur = jnp.where(iota == idx, _NEG, cur)
            vmax = jnp.where(i == 0, m, vmax)
            wexp = jnp.exp(m - vmax)
            hit = kiota == i
            idxs = jnp.where(hit, idx, idxs)
            wacc = jnp.where(hit, wexp, wacc)
            return cur, vmax, denom + wexp, idxs, wacc

        vmax0 = jnp.full((bb, 1), _NEG, jnp.float32)
        den0 = jnp.zeros((bb, 1), jnp.float32)
        idxs0 = jnp.zeros((bb, _LANES), jnp.int32)
        wacc0 = jnp.zeros((bb, _LANES), jnp.float32)
        _, _, denom, idxs, wacc = lax.fori_loop(
            0, k, body, (cur, vmax0, den0, idxs0, wacc0))

        brow = lax.broadcasted_iota(jnp.int32, (bb, _LANES), 0)
        idx_ref[...] = idxs + (brow + b_i * bb) * n_total
        w_ref[...] = jnp.where(kiota < k, wacc / denom, 0.0)


def _pool_head_kernel(rows_ref, w_ref, w_enc_ref, b_enc_ref, gamma_ref,
                      beta_ref, mean_ref, var_ref, w_head_ref, b_head_ref,
                      out_ref, *, btot, kpad):
    rows = rows_ref[...]
    z = w_enc_ref.shape[-1]
    emb = jnp.dot(rows, w_enc_ref[...], preferred_element_type=jnp.float32)
    emb = jnp.maximum(emb + b_enc_ref[...], 0.0)
    emb3 = emb.reshape(btot, kpad, z)
    w = w_ref[...]
    pooled = jnp.zeros((btot, z), jnp.float32)
    for i in range(kpad):
        pooled = pooled + w[:, i:i + 1] * emb3[:, i, :]
    bn = (pooled - mean_ref[...]) * lax.rsqrt(var_ref[...] + 1e-5)
    bn = bn * gamma_ref[...] + beta_ref[...]
    out = jnp.dot(bn, w_head_ref[...], preferred_element_type=jnp.float32)
    out_ref[...] = out + b_head_ref[...]


def _sc_gather(table, idx_flat):
    g_total, f = idx_flat.shape[0], table.shape[1]
    info = plsc.get_sparse_core_info()
    nw = info.num_cores * info.num_subcores
    g_per_w = g_total // nw
    mesh = plsc.VectorSubcoreMesh(core_axis_name="c", subcore_axis_name="s")

    @functools.partial(
        pl.kernel, mesh=mesh,
        out_type=jax.ShapeDtypeStruct((g_total, f), jnp.float32),
        scratch_types=[
            pltpu.VMEM((g_per_w,), jnp.int32),
            pltpu.VMEM((g_per_w, f), jnp.float32),
            pltpu.SemaphoreType.DMA,
        ],
    )
    def gather_kernel(table_hbm, idx_hbm, out_hbm, idx_v, rows_v, sem):
        wid = lax.axis_index("s") * info.num_cores + lax.axis_index("c")
        base = wid * g_per_w
        pltpu.sync_copy(idx_hbm.at[pl.ds(base, g_per_w)], idx_v)
        pltpu.async_copy(table_hbm.at[idx_v], rows_v, sem).wait()
        pltpu.sync_copy(rows_v, out_hbm.at[pl.ds(base, g_per_w)])

    return gather_kernel(table, idx_flat)


def kernel(bags, W_enc, b_enc, W_att, b_att, bn_gamma, bn_beta, bn_mean,
           bn_var, W_head, b_head):
    B, N, F = bags.shape
    Z = W_enc.shape[1]
    NOUT = W_head.shape[1]
    k = min(_K, N)

    BB = 8 if B % 8 == 0 else B
    NB = 4096 if N % 4096 == 0 else N
    num_n = N // NB

    b_enc2 = b_enc.reshape(1, Z)
    w_att2 = jnp.zeros((Z, _LANES), jnp.float32).at[:, 0:1].set(W_att)
    b_att2 = b_att.reshape(1, 1)

    score_body = functools.partial(_score_topk_kernel, bb=BB, nb_blk=NB,
                                   n_total=N, k=k)
    idx_out, w_out = pl.pallas_call(
        score_body,
        grid=(B // BB, num_n),
        in_specs=[
            pl.BlockSpec((BB, NB, F), lambda b, n: (b, n, 0)),
            pl.BlockSpec((F, Z), lambda b, n: (0, 0)),
            pl.BlockSpec((1, Z), lambda b, n: (0, 0)),
            pl.BlockSpec((Z, _LANES), lambda b, n: (0, 0)),
            pl.BlockSpec((1, 1), lambda b, n: (0, 0)),
        ],
        out_specs=[
            pl.BlockSpec((BB, _LANES), lambda b, n: (b, 0)),
            pl.BlockSpec((BB, _LANES), lambda b, n: (b, 0)),
        ],
        out_shape=[
            jax.ShapeDtypeStruct((B, _LANES), jnp.int32),
            jax.ShapeDtypeStruct((B, _LANES), jnp.float32),
        ],
        scratch_shapes=[
            pltpu.VMEM((num_n, BB, NB), jnp.float32),
        ],
        compiler_params=pltpu.CompilerParams(
            vmem_limit_bytes=100 * 1024 * 1024),
    )(bags, W_enc, b_enc2, w_att2, b_att2)

    idx_flat = idx_out[:, :_KPAD].reshape(B * _KPAD)
    rows = _sc_gather(bags.reshape(B * N, F), idx_flat)

    gamma2 = bn_gamma.reshape(1, Z)
    beta2 = bn_beta.reshape(1, Z)
    mean2 = bn_mean.reshape(1, Z)
    var2 = bn_var.reshape(1, Z)
    w_head_p = jnp.zeros((Z, _LANES), jnp.float32).at[:, :NOUT].set(W_head)
    b_head_p = jnp.zeros((1, _LANES), jnp.float32).at[:, :NOUT].set(b_head)

    pool_body = functools.partial(_pool_head_kernel, btot=B, kpad=_KPAD)
    out = pl.pallas_call(
        pool_body,
        out_shape=jax.ShapeDtypeStruct((B, _LANES), jnp.float32),
    )(rows, w_out, W_enc, b_enc2, gamma2, beta2, mean2, var2,
      w_head_p, b_head_p)
    return out[:, :NOUT]


# speedup vs baseline: 1.2150x; 1.2150x over previous
import functools

import jax
import jax.numpy as jnp
from jax import lax
from jax.experimental import pallas as pl
from jax.experimental.pallas import tpu as pltpu
from jax.experimental.pallas import tpu_sc as plsc

_K = 20
_KPAD = 32
_NEG = -3.0e38
_LANES = 128


def _score_topk_kernel(bags_ref, w_enc_ref, b_enc_ref, w_att_ref, b_att_ref,
                       idx_ref, w_ref, scores_ref, *, bb, nb_blk, n_total, k):
    b_i = pl.program_id(0)
    n_i = pl.program_id(1)
    num_b = pl.num_programs(0)
    num_n = pl.num_programs(1)

    x = bags_ref[...]
    f = x.shape[-1]
    x2 = x.reshape(bb * nb_blk, f)
    emb = jnp.dot(x2, w_enc_ref[...], preferred_element_type=jnp.float32)
    emb = jnp.maximum(emb + b_enc_ref[...], 0.0)
    s = jnp.dot(emb, w_att_ref[...],
                preferred_element_type=jnp.float32)[:, 0] + b_att_ref[0, 0]
    scores_ref[n_i, pl.ds(b_i * bb, bb), :] = s.reshape(bb, nb_blk)

    @pl.when(jnp.logical_and(b_i == num_b - 1, n_i == num_n - 1))
    def _finalize():
        btot = bb * num_b
        cur = jnp.concatenate([scores_ref[c] for c in range(num_n)],
                              axis=1)
        iota = lax.broadcasted_iota(jnp.int32, (btot, n_total), 1)
        kiota = lax.broadcasted_iota(jnp.int32, (btot, _LANES), 1)

        def body(i, carry):
            cur, vmax, denom, idxs, wacc = carry
            m = jnp.max(cur, axis=1, keepdims=True)
            cand = jnp.where(cur == m, iota, n_total)
            idx = jnp.min(cand, axis=1, keepdims=True)
            cur = jnp.where(iota == idx, _NEG, cur)
            vmax = jnp.where(i == 0, m, vmax)
            wexp = jnp.exp(m - vmax)
            hit = kiota == i
            idxs = jnp.where(hit, idx, idxs)
            wacc = jnp.where(hit, wexp, wacc)
            return cur, vmax, denom + wexp, idxs, wacc

        vmax0 = jnp.full((btot, 1), _NEG, jnp.float32)
        den0 = jnp.zeros((btot, 1), jnp.float32)
        idxs0 = jnp.zeros((btot, _LANES), jnp.int32)
        wacc0 = jnp.zeros((btot, _LANES), jnp.float32)
        _, _, denom, idxs, wacc = lax.fori_loop(
            0, k, body, (cur, vmax0, den0, idxs0, wacc0))

        brow = lax.broadcasted_iota(jnp.int32, (btot, _LANES), 0)
        idx_ref[...] = idxs + brow * n_total
        w_ref[...] = jnp.where(kiota < k, wacc / denom, 0.0)


def _pool_head_kernel(rows_ref, w_ref, w_enc_ref, b_enc_ref, gamma_ref,
                      beta_ref, mean_ref, var_ref, w_head_ref, b_head_ref,
                      out_ref, *, btot, kpad):
    rows = rows_ref[...]
    z = w_enc_ref.shape[-1]
    emb = jnp.dot(rows, w_enc_ref[...], preferred_element_type=jnp.float32)
    emb = jnp.maximum(emb + b_enc_ref[...], 0.0)
    emb3 = emb.reshape(btot, kpad, z)
    w = w_ref[...]
    pooled = jnp.zeros((btot, z), jnp.float32)
    for i in range(kpad):
        pooled = pooled + w[:, i:i + 1] * emb3[:, i, :]
    bn = (pooled - mean_ref[...]) * lax.rsqrt(var_ref[...] + 1e-5)
    bn = bn * gamma_ref[...] + beta_ref[...]
    out = jnp.dot(bn, w_head_ref[...], preferred_element_type=jnp.float32)
    out_ref[...] = out + b_head_ref[...]


def _sc_gather(table, idx_flat):
    g_total, f = idx_flat.shape[0], table.shape[1]
    info = plsc.get_sparse_core_info()
    nw = info.num_cores * info.num_subcores
    g_per_w = g_total // nw
    mesh = plsc.VectorSubcoreMesh(core_axis_name="c", subcore_axis_name="s")

    @functools.partial(
        pl.kernel, mesh=mesh,
        out_type=jax.ShapeDtypeStruct((g_total, f), jnp.float32),
        scratch_types=[
            pltpu.VMEM((g_per_w,), jnp.int32),
            pltpu.VMEM((g_per_w, f), jnp.float32),
            pltpu.SemaphoreType.DMA,
        ],
    )
    def gather_kernel(table_hbm, idx_hbm, out_hbm, idx_v, rows_v, sem):
        wid = lax.axis_index("s") * info.num_cores + lax.axis_index("c")
        base = wid * g_per_w
        pltpu.sync_copy(idx_hbm.at[pl.ds(base, g_per_w)], idx_v)
        pltpu.async_copy(table_hbm.at[idx_v], rows_v, sem).wait()
        pltpu.sync_copy(rows_v, out_hbm.at[pl.ds(base, g_per_w)])

    return gather_kernel(table, idx_flat)


def kernel(bags, W_enc, b_enc, W_att, b_att, bn_gamma, bn_beta, bn_mean,
           bn_var, W_head, b_head):
    B, N, F = bags.shape
    Z = W_enc.shape[1]
    NOUT = W_head.shape[1]
    k = min(_K, N)

    BB = 8 if B % 8 == 0 else B
    NB = 4096 if N % 4096 == 0 else N
    num_n = N // NB

    b_enc2 = b_enc.reshape(1, Z)
    w_att2 = jnp.zeros((Z, _LANES), jnp.float32).at[:, 0:1].set(W_att)
    b_att2 = b_att.reshape(1, 1)

    score_body = functools.partial(_score_topk_kernel, bb=BB, nb_blk=NB,
                                   n_total=N, k=k)
    idx_out, w_out = pl.pallas_call(
        score_body,
        grid=(B // BB, num_n),
        in_specs=[
            pl.BlockSpec((BB, NB, F), lambda b, n: (b, n, 0)),
            pl.BlockSpec((F, Z), lambda b, n: (0, 0)),
            pl.BlockSpec((1, Z), lambda b, n: (0, 0)),
            pl.BlockSpec((Z, _LANES), lambda b, n: (0, 0)),
            pl.BlockSpec((1, 1), lambda b, n: (0, 0)),
        ],
        out_specs=[
            pl.BlockSpec((B, _LANES), lambda b, n: (0, 0)),
            pl.BlockSpec((B, _LANES), lambda b, n: (0, 0)),
        ],
        out_shape=[
            jax.ShapeDtypeStruct((B, _LANES), jnp.int32),
            jax.ShapeDtypeStruct((B, _LANES), jnp.float32),
        ],
        scratch_shapes=[
            pltpu.VMEM((num_n, B, NB), jnp.float32),
        ],
        compiler_params=pltpu.CompilerParams(
            vmem_limit_bytes=100 * 1024 * 1024),
    )(bags, W_enc, b_enc2, w_att2, b_att2)

    idx_flat = idx_out[:, :_KPAD].reshape(B * _KPAD)
    rows = _sc_gather(bags.reshape(B * N, F), idx_flat)

    gamma2 = bn_gamma.reshape(1, Z)
    beta2 = bn_beta.reshape(1, Z)
    mean2 = bn_mean.reshape(1, Z)
    var2 = bn_var.reshape(1, Z)
    w_head_p = jnp.zeros((Z, _LANES), jnp.float32).at[:, :NOUT].set(W_head)
    b_head_p = jnp.zeros((1, _LANES), jnp.float32).at[:, :NOUT].set(b_head)

    pool_body = functools.partial(_pool_head_kernel, btot=B, kpad=_KPAD)
    out = pl.pallas_call(
        pool_body,
        out_shape=jax.ShapeDtypeStruct((B, _LANES), jnp.float32),
    )(rows, w_out, W_enc, b_enc2, gamma2, beta2, mean2, var2,
      w_head_p, b_head_p)
    return out[:, :NOUT]


# speedup vs baseline: 1.2234x; 1.0069x over previous
import functools

import jax
import jax.numpy as jnp
from jax import lax
from jax.experimental import pallas as pl
from jax.experimental.pallas import tpu as pltpu
from jax.experimental.pallas import tpu_sc as plsc

_K = 20
_KPAD = 32
_NEG = -3.0e38
_LANES = 128


def _score_topk_kernel(bags_ref, w_enc_ref, b_enc_ref, w_att_ref, b_att_ref,
                       idx_ref, w_ref, scores_ref, *, bb, nb_blk, n_total, k):
    b_i = pl.program_id(0)
    n_i = pl.program_id(1)
    num_b = pl.num_programs(0)
    num_n = pl.num_programs(1)

    x = bags_ref[...]
    f = x.shape[-1]
    x2 = x.reshape(bb * nb_blk, f)
    emb = jnp.dot(x2, w_enc_ref[...], preferred_element_type=jnp.float32)
    emb = jnp.maximum(emb + b_enc_ref[...], 0.0)
    s = jnp.dot(emb, w_att_ref[...],
                preferred_element_type=jnp.float32)[:, 0] + b_att_ref[0, 0]
    scores_ref[n_i, pl.ds(b_i * bb, bb), :] = s.reshape(bb, nb_blk)

    @pl.when(jnp.logical_and(b_i == num_b - 1, n_i == num_n - 1))
    def _finalize():
        btot = bb * num_b
        cur = jnp.concatenate([scores_ref[c] for c in range(num_n)],
                              axis=1)
        iota = lax.broadcasted_iota(jnp.int32, (btot, n_total), 1)
        kiota = lax.broadcasted_iota(jnp.int32, (btot, _LANES), 1)

        def body(i, carry):
            cur, vmax, denom, idxs, wacc = carry
            m = jnp.max(cur, axis=1, keepdims=True)
            cand = jnp.where(cur == m, iota, n_total)
            idx = jnp.min(cand, axis=1, keepdims=True)
            cur = jnp.where(iota == idx, _NEG, cur)
            vmax = jnp.where(i == 0, m, vmax)
            wexp = jnp.exp(m - vmax)
            hit = kiota == i
            idxs = jnp.where(hit, idx, idxs)
            wacc = jnp.where(hit, wexp, wacc)
            return cur, vmax, denom + wexp, idxs, wacc

        vmax0 = jnp.full((btot, 1), _NEG, jnp.float32)
        den0 = jnp.zeros((btot, 1), jnp.float32)
        idxs0 = jnp.zeros((btot, _LANES), jnp.int32)
        wacc0 = jnp.zeros((btot, _LANES), jnp.float32)
        _, _, denom, idxs, wacc = lax.fori_loop(
            0, k, body, (cur, vmax0, den0, idxs0, wacc0))

        brow = lax.broadcasted_iota(jnp.int32, (btot, _LANES), 0)
        idx_ref[...] = idxs + brow * n_total
        w_ref[...] = jnp.where(kiota < k, wacc / denom, 0.0)


def _pool_head_kernel(rows_ref, w_ref, w_enc_ref, b_enc_ref, gamma_ref,
                      beta_ref, mean_ref, var_ref, w_head_ref, b_head_ref,
                      out_ref, *, btot, kpad):
    rows = rows_ref[...]
    z = w_enc_ref.shape[-1]
    emb = jnp.dot(rows, w_enc_ref[...], preferred_element_type=jnp.float32)
    emb = jnp.maximum(emb + b_enc_ref[...], 0.0)
    emb3 = emb.reshape(btot, kpad, z)
    w = w_ref[...]
    pooled = jnp.zeros((btot, z), jnp.float32)
    for i in range(kpad):
        pooled = pooled + w[:, i:i + 1] * emb3[:, i, :]
    bn = (pooled - mean_ref[...]) * lax.rsqrt(var_ref[...] + 1e-5)
    bn = bn * gamma_ref[...] + beta_ref[...]
    out = jnp.dot(bn, w_head_ref[...], preferred_element_type=jnp.float32)
    out_ref[...] = out + b_head_ref[...]


def _sc_gather(table, idx_flat):
    g_total, f = idx_flat.shape[0], table.shape[1]
    info = plsc.get_sparse_core_info()
    nw = info.num_cores * info.num_subcores
    g_per_w = g_total // nw
    mesh = plsc.VectorSubcoreMesh(core_axis_name="c", subcore_axis_name="s")

    @functools.partial(
        pl.kernel, mesh=mesh,
        out_type=jax.ShapeDtypeStruct((g_total, f), jnp.float32),
        scratch_types=[
            pltpu.VMEM((g_per_w,), jnp.int32),
            pltpu.VMEM((g_per_w, f), jnp.float32),
            pltpu.SemaphoreType.DMA,
        ],
    )
    def gather_kernel(table_hbm, idx_hbm, out_hbm, idx_v, rows_v, sem):
        wid = lax.axis_index("s") * info.num_cores + lax.axis_index("c")
        base = wid * g_per_w
        pltpu.sync_copy(idx_hbm.at[pl.ds(base, g_per_w)], idx_v)
        pltpu.async_copy(table_hbm.at[idx_v], rows_v, sem).wait()
        pltpu.sync_copy(rows_v, out_hbm.at[pl.ds(base, g_per_w)])

    return gather_kernel(table, idx_flat)


def kernel(bags, W_enc, b_enc, W_att, b_att, bn_gamma, bn_beta, bn_mean,
           bn_var, W_head, b_head):
    B, N, F = bags.shape
    Z = W_enc.shape[1]
    NOUT = W_head.shape[1]
    k = min(_K, N)

    BB = 8 if B % 8 == 0 else B
    NB = 4096 if N % 4096 == 0 else N
    num_n = N // NB

    b_enc2 = b_enc.reshape(1, Z)
    w_att2 = jnp.zeros((Z, _LANES), jnp.float32).at[:, 0:1].set(W_att)
    b_att2 = b_att.reshape(1, 1)

    score_body = functools.partial(_score_topk_kernel, bb=BB, nb_blk=NB,
                                   n_total=N, k=k)
    idx_out, w_out = pl.pallas_call(
        score_body,
        grid=(B // BB, num_n),
        in_specs=[
            pl.BlockSpec((BB, NB, F), lambda b, n: (b, n, 0)),
            pl.BlockSpec((F, Z), lambda b, n: (0, 0)),
            pl.BlockSpec((1, Z), lambda b, n: (0, 0)),
            pl.BlockSpec((Z, _LANES), lambda b, n: (0, 0)),
            pl.BlockSpec((1, 1), lambda b, n: (0, 0)),
        ],
        out_specs=[
            pl.BlockSpec((B, _LANES), lambda b, n: (0, 0)),
            pl.BlockSpec((B, _LANES), lambda b, n: (0, 0)),
        ],
        out_shape=[
            jax.ShapeDtypeStruct((B, _LANES), jnp.int32),
            jax.ShapeDtypeStruct((B, _LANES), jnp.float32),
        ],
        scratch_shapes=[
            pltpu.VMEM((num_n, B, NB), jnp.float32),
        ],
        compiler_params=pltpu.CompilerParams(
            vmem_limit_bytes=100 * 1024 * 1024),
    )(bags, W_enc, b_enc2, w_att2, b_att2)

    kg = k if (B * k) % 256 == 0 else _KPAD
    idx_flat = idx_out[:, :kg].reshape(B * kg)
    rows = _sc_gather(bags.reshape(B * N, F), idx_flat)

    gamma2 = bn_gamma.reshape(1, Z)
    beta2 = bn_beta.reshape(1, Z)
    mean2 = bn_mean.reshape(1, Z)
    var2 = bn_var.reshape(1, Z)
    w_head_p = jnp.zeros((Z, _LANES), jnp.float32).at[:, :NOUT].set(W_head)
    b_head_p = jnp.zeros((1, _LANES), jnp.float32).at[:, :NOUT].set(b_head)

    pool_body = functools.partial(_pool_head_kernel, btot=B, kpad=kg)
    out = pl.pallas_call(
        pool_body,
        out_shape=jax.ShapeDtypeStruct((B, _LANES), jnp.float32),
    )(rows, w_out, W_enc, b_enc2, gamma2, beta2, mean2, var2,
      w_head_p, b_head_p)
    return out[:, :NOUT]
